# baseline (speedup 1.0000x reference)
import functools
import math

import jax
import jax.numpy as jnp
import numpy as np
from jax import lax
from jax.experimental import pallas as pl
from jax.experimental.pallas import tpu as pltpu
from jax.experimental.pallas import tpu_sc as plsc

IN = 64
HID = 64
NU = 128
TOPK = 8
KS = 64
QS = 64
VS = 64
RB = 512
NW = 32
LANES = 16


def _scores_body(x_ref, hs_ref, key_w_ref, qw_ref, st_ref):
    q = jnp.sum(hs_ref[...][:, :, None] * qw_ref[...], axis=1)
    k = jax.lax.dot_general(x_ref[...], key_w_ref[...],
                            (((1,), (0,)), ((), ())),
                            preferred_element_type=jnp.float32)
    st = jax.lax.dot_general(q, k, (((1,), (1,)), ((), ())),
                             preferred_element_type=jnp.float32)
    st_ref[...] = st * (1.0 / math.sqrt(KS))


def _topk_mask_body(st_hbm, mt_hbm, sv, mv):
    cpw = st_hbm.shape[1] // NW
    wid = lax.axis_index("s") * 2 + lax.axis_index("c")
    base = wid * cpw
    pltpu.sync_copy(st_hbm.at[:, pl.ds(base, cpw)], sv)

    ones_i = jnp.full((LANES,), 1, jnp.int32)
    zero_i = jnp.full((LANES,), 0, jnp.int32)
    topk_i = jnp.full((LANES,), TOPK, jnp.int32)
    one_f = jnp.full((LANES,), 1.0, jnp.float32)
    zero_f = jnp.full((LANES,), 0.0, jnp.float32)
    neg_inf = jnp.full((LANES,), -3.0e38, jnp.float32)

    for g in range(cpw // LANES):
        col = g * LANES

        def pass1(u, m):
            new = sv[u, pl.ds(col, LANES)]
            out = []
            for j in range(TOPK):
                hi = jnp.maximum(m[j], new)
                new = jnp.minimum(m[j], new)
                out.append(hi)
            return tuple(out)
        m = lax.fori_loop(0, NU, pass1, (neg_inf,) * TOPK)
        t = m[TOPK - 1]

        def pass2(u, cnt):
            gt = sv[u, pl.ds(col, LANES)] > t
            return cnt + jnp.where(gt, ones_i, zero_i)
        cnt = lax.fori_loop(0, NU, pass2, zero_i)
        quota = topk_i - cnt

        def pass3(u, q):
            s_u = sv[u, pl.ds(col, LANES)]
            take_eq = (s_u == t) & (q > zero_i)
            sel = (s_u > t) | take_eq
            w_u = jnp.where(sel, one_f / (one_f + jnp.exp(zero_f - s_u)),
                            zero_f)
            mv[u, pl.ds(col, LANES)] = w_u
            return q - jnp.where(take_eq, ones_i, zero_i)
        lax.fori_loop(0, NU, pass3, quota)

    pltpu.sync_copy(mv, mt_hbm.at[:, pl.ds(base, cpw)])


def _value_body(x_ref, mt_ref, w3_ref, e_ref, out_ref):
    xb = x_ref[...]
    w = jnp.transpose(mt_ref[...])
    v = jax.lax.dot_general(xb.astype(jnp.bfloat16), w3_ref[...],
                            (((1,), (0,)), ((), ())),
                            preferred_element_type=jnp.float32)
    p = v * jnp.tile(w, (1, VS))
    out_ref[...] = jax.lax.dot_general(p, e_ref[...], (((1,), (0,)), ((), ())),
                                       preferred_element_type=jnp.float32)


def kernel(x, hs, key_w, key_b, hs_value_w, query_w):
    del key_b
    b = x.shape[0]
    x2 = x.reshape(b, IN)

    st = pl.pallas_call(
        _scores_body,
        in_specs=[
            pl.BlockSpec((b, IN), lambda: (0, 0)),
            pl.BlockSpec((NU, HID), lambda: (0, 0)),
            pl.BlockSpec((IN, KS), lambda: (0, 0)),
            pl.BlockSpec((NU, HID, QS), lambda: (0, 0, 0)),
        ],
        out_specs=pl.BlockSpec((NU, b), lambda: (0, 0)),
        out_shape=jax.ShapeDtypeStruct((NU, b), jnp.float32),
    )(x2, hs, key_w, query_w)

    cpw = b // NW
    topk_mask = functools.partial(
        pl.kernel,
        mesh=plsc.VectorSubcoreMesh(core_axis_name="c", subcore_axis_name="s"),
        out_type=jax.ShapeDtypeStruct((NU, b), jnp.float32),
        scratch_types=[
            pltpu.VMEM((NU, cpw), jnp.float32),
            pltpu.VMEM((NU, cpw), jnp.float32),
        ],
    )(_topk_mask_body)
    mt = topk_mask(st)

    w3 = jnp.transpose(hs_value_w, (1, 2, 0)).reshape(IN, VS * NU)
    w3 = w3.astype(jnp.bfloat16)
    e = jnp.asarray((np.arange(VS * NU)[:, None] // NU
                     == np.arange(VS)[None, :]).astype(np.float32))
    out = pl.pallas_call(
        _value_body,
        grid=(b // RB,),
        in_specs=[
            pl.BlockSpec((RB, IN), lambda i: (i, 0)),
            pl.BlockSpec((NU, RB), lambda i: (0, i)),
            pl.BlockSpec((IN, VS * NU), lambda i: (0, 0)),
            pl.BlockSpec((VS * NU, VS), lambda i: (0, 0)),
        ],
        out_specs=pl.BlockSpec((RB, VS), lambda i: (i, 0)),
        out_shape=jax.ShapeDtypeStruct((b, VS), jnp.float32),
    )(x2, mt, w3, e)
    return out


# speedup vs baseline: 1.0057x; 1.0057x over previous
import functools
import math

import jax
import jax.numpy as jnp
import numpy as np
from jax import lax
from jax.experimental import pallas as pl
from jax.experimental.pallas import tpu as pltpu
from jax.experimental.pallas import tpu_sc as plsc

IN = 64
HID = 64
NU = 128
TOPK = 8
KS = 64
QS = 64
VS = 64
RB = 512
NW = 32
LANES = 16


def _scores_body(x_ref, hs_ref, key_w_ref, qw_ref, st_ref):
    q = jnp.sum(hs_ref[...][:, :, None] * qw_ref[...], axis=1)
    k = jax.lax.dot_general(x_ref[...], key_w_ref[...],
                            (((1,), (0,)), ((), ())),
                            preferred_element_type=jnp.float32)
    st = jax.lax.dot_general(q, k, (((1,), (1,)), ((), ())),
                             preferred_element_type=jnp.float32)
    st_ref[...] = st * (1.0 / math.sqrt(KS))


def _topk_mask_body(st_hbm, mt_hbm, sv, mv):
    cpw = st_hbm.shape[1] // NW
    wid = lax.axis_index("s") * 2 + lax.axis_index("c")
    base = wid * cpw
    pltpu.sync_copy(st_hbm.at[:, pl.ds(base, cpw)], sv)

    ones_i = jnp.full((LANES,), 1, jnp.int32)
    zero_i = jnp.full((LANES,), 0, jnp.int32)
    topk_i = jnp.full((LANES,), TOPK, jnp.int32)
    one_f = jnp.full((LANES,), 1.0, jnp.float32)
    zero_f = jnp.full((LANES,), 0.0, jnp.float32)
    neg_inf = jnp.full((LANES,), -3.0e38, jnp.float32)

    for g in range(cpw // LANES):
        col = g * LANES

        def pass1(u, m):
            new = sv[u, pl.ds(col, LANES)]
            out = []
            for j in range(TOPK):
                hi = jnp.maximum(m[j], new)
                new = jnp.minimum(m[j], new)
                out.append(hi)
            return tuple(out)
        m = lax.fori_loop(0, NU, pass1, (neg_inf,) * TOPK)
        t = m[TOPK - 1]

        def pass2(u, cnt):
            gt = sv[u, pl.ds(col, LANES)] > t
            return cnt + jnp.where(gt, ones_i, zero_i)
        cnt = lax.fori_loop(0, NU, pass2, zero_i)
        quota = topk_i - cnt

        def pass3(u, q):
            s_u = sv[u, pl.ds(col, LANES)]
            take_eq = (s_u == t) & (q > zero_i)
            sel = (s_u > t) | take_eq
            w_u = jnp.where(sel, one_f / (one_f + jnp.exp(zero_f - s_u)),
                            zero_f)
            mv[u, pl.ds(col, LANES)] = w_u
            return q - jnp.where(take_eq, ones_i, zero_i)
        lax.fori_loop(0, NU, pass3, quota)

    pltpu.sync_copy(mv, mt_hbm.at[:, pl.ds(base, cpw)])


def _value_body(x_ref, mt_ref, w3_ref, e_ref, out_ref):
    xb = x_ref[...]
    w = jnp.transpose(mt_ref[...])
    v = jax.lax.dot_general(xb.astype(jnp.bfloat16), w3_ref[...],
                            (((1,), (0,)), ((), ())),
                            preferred_element_type=jnp.float32)
    p = (v * jnp.tile(w, (1, VS))).astype(jnp.bfloat16)
    out_ref[...] = jax.lax.dot_general(p, e_ref[...], (((1,), (0,)), ((), ())),
                                       preferred_element_type=jnp.float32)


def kernel(x, hs, key_w, key_b, hs_value_w, query_w):
    del key_b
    b = x.shape[0]
    x2 = x.reshape(b, IN)

    st = pl.pallas_call(
        _scores_body,
        in_specs=[
            pl.BlockSpec((b, IN), lambda: (0, 0)),
            pl.BlockSpec((NU, HID), lambda: (0, 0)),
            pl.BlockSpec((IN, KS), lambda: (0, 0)),
            pl.BlockSpec((NU, HID, QS), lambda: (0, 0, 0)),
        ],
        out_specs=pl.BlockSpec((NU, b), lambda: (0, 0)),
        out_shape=jax.ShapeDtypeStruct((NU, b), jnp.float32),
    )(x2, hs, key_w, query_w)

    cpw = b // NW
    topk_mask = functools.partial(
        pl.kernel,
        mesh=plsc.VectorSubcoreMesh(core_axis_name="c", subcore_axis_name="s"),
        out_type=jax.ShapeDtypeStruct((NU, b), jnp.float32),
        scratch_types=[
            pltpu.VMEM((NU, cpw), jnp.float32),
            pltpu.VMEM((NU, cpw), jnp.float32),
        ],
    )(_topk_mask_body)
    mt = topk_mask(st)

    w3 = jnp.transpose(hs_value_w, (1, 2, 0)).reshape(IN, VS * NU)
    w3 = w3.astype(jnp.bfloat16)
    e = jnp.asarray((np.arange(VS * NU)[:, None] // NU
                     == np.arange(VS)[None, :]).astype(np.float32)
                    ).astype(jnp.bfloat16)
    out = pl.pallas_call(
        _value_body,
        grid=(b // RB,),
        in_specs=[
            pl.BlockSpec((RB, IN), lambda i: (i, 0)),
            pl.BlockSpec((NU, RB), lambda i: (0, i)),
            pl.BlockSpec((IN, VS * NU), lambda i: (0, 0)),
            pl.BlockSpec((VS * NU, VS), lambda i: (0, 0)),
        ],
        out_specs=pl.BlockSpec((RB, VS), lambda i: (i, 0)),
        out_shape=jax.ShapeDtypeStruct((b, VS), jnp.float32),
    )(x2, mt, w3, e)
    return out


# speedup vs baseline: 1.0481x; 1.0421x over previous
import functools
import math

import jax
import jax.numpy as jnp
import numpy as np
from jax import lax
from jax.experimental import pallas as pl
from jax.experimental.pallas import tpu as pltpu
from jax.experimental.pallas import tpu_sc as plsc

IN = 64
HID = 64
NU = 128
TOPK = 8
KS = 64
QS = 64
VS = 64
RB = 512
NW = 32
LANES = 16


def _scores_body(x_ref, hs_ref, key_w_ref, qw_ref, st_ref):
    q = jnp.sum(hs_ref[...][:, :, None] * qw_ref[...], axis=1)
    k = jax.lax.dot_general(x_ref[...], key_w_ref[...],
                            (((1,), (0,)), ((), ())),
                            preferred_element_type=jnp.float32)
    st = jax.lax.dot_general(q, k, (((1,), (1,)), ((), ())),
                             preferred_element_type=jnp.float32)
    st_ref[...] = st * (1.0 / math.sqrt(KS))


def _topk_mask_body(st_hbm, mt_hbm, sv, mv):
    cpw = st_hbm.shape[1] // NW
    wid = lax.axis_index("s") * 2 + lax.axis_index("c")
    base = wid * cpw
    pltpu.sync_copy(st_hbm.at[:, pl.ds(base, cpw)], sv)

    ones_i = jnp.full((LANES,), 1, jnp.int32)
    zero_i = jnp.full((LANES,), 0, jnp.int32)
    topk_i = jnp.full((LANES,), TOPK, jnp.int32)
    one_f = jnp.full((LANES,), 1.0, jnp.float32)
    zero_f = jnp.full((LANES,), 0.0, jnp.float32)
    neg_inf = jnp.full((LANES,), -3.0e38, jnp.float32)

    for g in range(cpw // LANES):
        col = g * LANES

        def pass1(u, m):
            new = sv[u, pl.ds(col, LANES)]
            out = []
            for j in range(TOPK):
                hi = jnp.maximum(m[j], new)
                new = jnp.minimum(m[j], new)
                out.append(hi)
            return tuple(out)
        m = lax.fori_loop(0, NU, pass1, (neg_inf,) * TOPK)
        t = m[TOPK - 1]

        cnt = zero_i
        for j in range(TOPK - 1):
            cnt = cnt + jnp.where(m[j] > t, ones_i, zero_i)
        quota = topk_i - cnt

        def pass3(u, q):
            s_u = sv[u, pl.ds(col, LANES)]
            take_eq = (s_u == t) & (q > zero_i)
            sel = (s_u > t) | take_eq
            w_u = jnp.where(sel, one_f / (one_f + jnp.exp(zero_f - s_u)),
                            zero_f)
            mv[u, pl.ds(col, LANES)] = w_u
            return q - jnp.where(take_eq, ones_i, zero_i)
        lax.fori_loop(0, NU, pass3, quota)

    pltpu.sync_copy(mv, mt_hbm.at[:, pl.ds(base, cpw)])


def _value_body(x_ref, mt_ref, w3_ref, e_ref, out_ref):
    xb = x_ref[...]
    w = jnp.transpose(mt_ref[...]).astype(jnp.bfloat16)
    v = jax.lax.dot_general(xb.astype(jnp.bfloat16), w3_ref[...],
                            (((1,), (0,)), ((), ())),
                            preferred_element_type=jnp.float32)
    p = v.astype(jnp.bfloat16) * jnp.tile(w, (1, VS))
    out_ref[...] = jax.lax.dot_general(p, e_ref[...], (((1,), (0,)), ((), ())),
                                       preferred_element_type=jnp.float32)


def kernel(x, hs, key_w, key_b, hs_value_w, query_w):
    del key_b
    b = x.shape[0]
    x2 = x.reshape(b, IN)

    st = pl.pallas_call(
        _scores_body,
        in_specs=[
            pl.BlockSpec((b, IN), lambda: (0, 0)),
            pl.BlockSpec((NU, HID), lambda: (0, 0)),
            pl.BlockSpec((IN, KS), lambda: (0, 0)),
            pl.BlockSpec((NU, HID, QS), lambda: (0, 0, 0)),
        ],
        out_specs=pl.BlockSpec((NU, b), lambda: (0, 0)),
        out_shape=jax.ShapeDtypeStruct((NU, b), jnp.float32),
    )(x2, hs, key_w, query_w)

    cpw = b // NW
    topk_mask = functools.partial(
        pl.kernel,
        mesh=plsc.VectorSubcoreMesh(core_axis_name="c", subcore_axis_name="s"),
        out_type=jax.ShapeDtypeStruct((NU, b), jnp.float32),
        scratch_types=[
            pltpu.VMEM((NU, cpw), jnp.float32),
            pltpu.VMEM((NU, cpw), jnp.float32),
        ],
    )(_topk_mask_body)
    mt = topk_mask(st)

    w3 = jnp.transpose(hs_value_w, (1, 2, 0)).reshape(IN, VS * NU)
    w3 = w3.astype(jnp.bfloat16)
    e = jnp.asarray((np.arange(VS * NU)[:, None] // NU
                     == np.arange(VS)[None, :]).astype(np.float32)
                    ).astype(jnp.bfloat16)
    out = pl.pallas_call(
        _value_body,
        grid=(b // RB,),
        in_specs=[
            pl.BlockSpec((RB, IN), lambda i: (i, 0)),
            pl.BlockSpec((NU, RB), lambda i: (0, i)),
            pl.BlockSpec((IN, VS * NU), lambda i: (0, 0)),
            pl.BlockSpec((VS * NU, VS), lambda i: (0, 0)),
        ],
        out_specs=pl.BlockSpec((RB, VS), lambda i: (i, 0)),
        out_shape=jax.ShapeDtypeStruct((b, VS), jnp.float32),
    )(x2, mt, w3, e)
    return out


# speedup vs baseline: 1.0561x; 1.0077x over previous
import functools
import math

import jax
import jax.numpy as jnp
import numpy as np
from jax import lax
from jax.experimental import pallas as pl
from jax.experimental.pallas import tpu as pltpu
from jax.experimental.pallas import tpu_sc as plsc

IN = 64
HID = 64
NU = 128
TOPK = 8
KS = 64
QS = 64
VS = 64
RB = 512
NW = 32
LANES = 16


def _scores_body(x_ref, hs_ref, key_w_ref, qw_ref, st_ref):
    q = jnp.sum(hs_ref[...][:, :, None] * qw_ref[...], axis=1)
    k = jax.lax.dot_general(x_ref[...], key_w_ref[...],
                            (((1,), (0,)), ((), ())),
                            preferred_element_type=jnp.float32)
    st = jax.lax.dot_general(q, k, (((1,), (1,)), ((), ())),
                             preferred_element_type=jnp.float32)
    st_ref[...] = st * (1.0 / math.sqrt(KS))


def _topk_mask_body(st_hbm, mt_hbm, sv, mv):
    cpw = st_hbm.shape[1] // NW
    wid = lax.axis_index("s") * 2 + lax.axis_index("c")
    base = wid * cpw
    pltpu.sync_copy(st_hbm.at[:, pl.ds(base, cpw)], sv)

    ones_i = jnp.full((LANES,), 1, jnp.int32)
    zero_i = jnp.full((LANES,), 0, jnp.int32)
    topk_i = jnp.full((LANES,), TOPK, jnp.int32)
    neg_inf = jnp.full((LANES,), -3.0e38, jnp.float32)

    for g in range(cpw // LANES):
        col = g * LANES

        def pass1(u, m):
            new = sv[u, pl.ds(col, LANES)]
            out = []
            for j in range(TOPK):
                hi = jnp.maximum(m[j], new)
                new = jnp.minimum(m[j], new)
                out.append(hi)
            return tuple(out)
        m = lax.fori_loop(0, NU, pass1, (neg_inf,) * TOPK)
        t = m[TOPK - 1]

        cnt = zero_i
        for j in range(TOPK - 1):
            cnt = cnt + jnp.where(m[j] > t, ones_i, zero_i)
        quota = topk_i - cnt

        def pass3(u, q):
            s_u = sv[u, pl.ds(col, LANES)]
            take_eq = (s_u == t) & (q > zero_i)
            sel = (s_u > t) | take_eq
            mv[u, pl.ds(col, LANES)] = jnp.where(sel, s_u, neg_inf)
            return q - jnp.where(take_eq, ones_i, zero_i)
        lax.fori_loop(0, NU, pass3, quota)

    pltpu.sync_copy(mv, mt_hbm.at[:, pl.ds(base, cpw)])


def _value_body(x_ref, mt_ref, w3_ref, e_ref, out_ref):
    xb = x_ref[...]
    sw = jnp.transpose(mt_ref[...])
    w = (1.0 / (1.0 + jnp.exp(-sw))).astype(jnp.bfloat16)
    v = jax.lax.dot_general(xb.astype(jnp.bfloat16), w3_ref[...],
                            (((1,), (0,)), ((), ())),
                            preferred_element_type=jnp.float32)
    p = v.astype(jnp.bfloat16) * jnp.tile(w, (1, VS))
    out_ref[...] = jax.lax.dot_general(p, e_ref[...], (((1,), (0,)), ((), ())),
                                       preferred_element_type=jnp.float32)


def kernel(x, hs, key_w, key_b, hs_value_w, query_w):
    del key_b
    b = x.shape[0]
    x2 = x.reshape(b, IN)

    st = pl.pallas_call(
        _scores_body,
        in_specs=[
            pl.BlockSpec((b, IN), lambda: (0, 0)),
            pl.BlockSpec((NU, HID), lambda: (0, 0)),
            pl.BlockSpec((IN, KS), lambda: (0, 0)),
            pl.BlockSpec((NU, HID, QS), lambda: (0, 0, 0)),
        ],
        out_specs=pl.BlockSpec((NU, b), lambda: (0, 0)),
        out_shape=jax.ShapeDtypeStruct((NU, b), jnp.float32),
    )(x2, hs, key_w, query_w)

    cpw = b // NW
    topk_mask = functools.partial(
        pl.kernel,
        mesh=plsc.VectorSubcoreMesh(core_axis_name="c", subcore_axis_name="s"),
        out_type=jax.ShapeDtypeStruct((NU, b), jnp.float32),
        scratch_types=[
            pltpu.VMEM((NU, cpw), jnp.float32),
            pltpu.VMEM((NU, cpw), jnp.float32),
        ],
    )(_topk_mask_body)
    mt = topk_mask(st)

    w3 = jnp.transpose(hs_value_w, (1, 2, 0)).reshape(IN, VS * NU)
    w3 = w3.astype(jnp.bfloat16)
    e = jnp.asarray((np.arange(VS * NU)[:, None] // NU
                     == np.arange(VS)[None, :]).astype(np.float32)
                    ).astype(jnp.bfloat16)
    out = pl.pallas_call(
        _value_body,
        grid=(b // RB,),
        in_specs=[
            pl.BlockSpec((RB, IN), lambda i: (i, 0)),
            pl.BlockSpec((NU, RB), lambda i: (0, i)),
            pl.BlockSpec((IN, VS * NU), lambda i: (0, 0)),
            pl.BlockSpec((VS * NU, VS), lambda i: (0, 0)),
        ],
        out_specs=pl.BlockSpec((RB, VS), lambda i: (i, 0)),
        out_shape=jax.ShapeDtypeStruct((b, VS), jnp.float32),
    )(x2, mt, w3, e)
    return out
